# Initial kernel scaffold; baseline (speedup 1.0000x reference)
import jax
import jax.numpy as jnp
from jax.experimental import pallas as pl


def kernel(x, edge_index, W1, b1, W2, b2):
    raise NotImplementedError("write your pallas kernel here")


# speedup vs baseline: 41.6000x; 41.6000x over previous
import functools

import jax
import jax.numpy as jnp
from jax import lax
from jax.experimental import pallas as pl
from jax.experimental.pallas import tpu as pltpu
from jax.experimental.pallas import tpu_sc as plsc

N_NODES = 10000
D_FEAT = 128
HIDDEN = 32
N_EDGES = 320000

NPAD = 10240
NC = 2
NS = 16
NW = NC * NS
CHUNK = 128
EPAD = 323584
NCH = EPAD // (NW * CHUNK)
NE_TILE = NCH * CHUNK
ROWS_T = NPAD // NS
PAD_NODE = N_NODES

_f32 = jnp.float32


def _sc_mesh():
    return plsc.VectorSubcoreMesh(core_axis_name="c", subcore_axis_name="s")


_SC_PARAMS = pltpu.CompilerParams(use_tc_tiling_on_sc=False,
                                  needs_layout_passes=False)


def _sc_scatter_rows(v, src3, dst3):
    @functools.partial(
        pl.kernel,
        out_type=jax.ShapeDtypeStruct((NC, NPAD, HIDDEN), _f32),
        mesh=_sc_mesh(),
        compiler_params=_SC_PARAMS,
        scratch_types=[
            pltpu.VMEM((NCH, CHUNK), jnp.int32),
            pltpu.VMEM((NCH, CHUNK), jnp.int32),
            pltpu.VMEM((CHUNK, HIDDEN), _f32),
            pltpu.VMEM_SHARED((NPAD, HIDDEN), _f32),
            pltpu.SemaphoreType.DMA,
        ],
    )
    def k(v_hbm, src_hbm, dst_hbm, out_hbm, src_v, dst_v, rows_v, acc_sh, sem):
        cid = lax.axis_index("c")
        sid = lax.axis_index("s")
        wid = sid * NC + cid
        r0 = sid * ROWS_T
        pltpu.sync_copy(src_hbm.at[wid], src_v)
        pltpu.sync_copy(dst_hbm.at[wid], dst_v)
        pltpu.sync_copy(v_hbm.at[pl.ds(r0, ROWS_T)], acc_sh.at[pl.ds(r0, ROWS_T)])
        plsc.subcore_barrier()

        def body(j, carry):
            pltpu.async_copy(v_hbm.at[src_v.at[j]], rows_v, sem).wait()
            pltpu.sync_copy(rows_v, acc_sh.at[dst_v.at[j]], add=True)
            return carry

        lax.fori_loop(0, NCH, body, 0, unroll=False)
        plsc.subcore_barrier()
        pltpu.sync_copy(acc_sh.at[pl.ds(r0, ROWS_T)],
                        out_hbm.at[cid, pl.ds(r0, ROWS_T)])

    return k(v, src3, dst3)


def _sc_scatter_scalar(v, srcf, dst3, do_gather):
    @functools.partial(
        pl.kernel,
        out_type=jax.ShapeDtypeStruct((NC, NPAD), _f32),
        mesh=_sc_mesh(),
        compiler_params=_SC_PARAMS,
        scratch_types=[
            pltpu.VMEM((NE_TILE,), jnp.int32),
            pltpu.VMEM((NCH, CHUNK), jnp.int32),
            pltpu.VMEM((NPAD,), _f32),
            pltpu.VMEM((NE_TILE,), _f32),
            pltpu.VMEM_SHARED((NPAD,), _f32),
            pltpu.SemaphoreType.DMA,
        ],
    )
    def k(v_hbm, src_hbm, dst_hbm, out_hbm, src_v, dst_v, g_v, val_v, acc_sh, sem):
        cid = lax.axis_index("c")
        sid = lax.axis_index("s")
        wid = sid * NC + cid
        r0 = sid * ROWS_T
        pltpu.sync_copy(dst_hbm.at[wid], dst_v)
        pltpu.sync_copy(v_hbm.at[pl.ds(r0, ROWS_T)], acc_sh.at[pl.ds(r0, ROWS_T)])
        if do_gather:
            pltpu.sync_copy(src_hbm.at[wid], src_v)
            pltpu.sync_copy(v_hbm, g_v)

            def gbody(i, carry):
                idx = src_v[pl.ds(i * 16, 16)]
                val_v[pl.ds(i * 16, 16)] = plsc.load_gather(g_v, [idx])
                return carry

            lax.fori_loop(0, NE_TILE // 16, gbody, 0, unroll=False)
        else:
            ones = jnp.full((16,), 1.0, dtype=_f32)
            for i in range(CHUNK // 16):
                val_v[pl.ds(i * 16, 16)] = ones
        plsc.subcore_barrier()

        def body(j, carry):
            if do_gather:
                src = val_v.at[pl.ds(j * CHUNK, CHUNK)]
            else:
                src = val_v.at[pl.ds(0, CHUNK)]
            pltpu.sync_copy(src, acc_sh.at[dst_v.at[j]], add=True)
            return carry

        lax.fori_loop(0, NCH, body, 0, unroll=False)
        plsc.subcore_barrier()
        pltpu.sync_copy(acc_sh.at[pl.ds(r0, ROWS_T)],
                        out_hbm.at[cid, pl.ds(r0, ROWS_T)])

    return k(v, srcf, dst3)


ROWS_B = 1280


def _tc_call(body, nout, *args):
    specs = []
    for a in args:
        if a.shape[0] == NPAD:
            specs.append(pl.BlockSpec((ROWS_B,) + a.shape[1:],
                                      lambda i: (i,) + (0,) * (a.ndim - 1)))
        else:
            specs.append(pl.BlockSpec(a.shape, lambda i: (0,) * a.ndim))
    out_shapes = [jax.ShapeDtypeStruct((NPAD, w), _f32) for w in nout]
    out_specs = [pl.BlockSpec((ROWS_B, w), lambda i: (i, 0)) for w in nout]
    res = pl.pallas_call(
        body,
        grid=(NPAD // ROWS_B,),
        in_specs=specs,
        out_specs=out_specs,
        out_shape=out_shapes,
    )(*args)
    return res


def _tc_deg_g1(x, W1, dp0, dp1):
    def body(x_ref, w_ref, d0_ref, d1_ref, dinv_ref, g1_ref):
        deg = d0_ref[...] + d1_ref[...] - 1.0
        dinv = lax.rsqrt(jnp.maximum(deg, 1.0))
        h0 = jnp.dot(x_ref[...], w_ref[...], preferred_element_type=_f32)
        dinv_ref[...] = dinv
        g1_ref[...] = dinv * h0

    return _tc_call(body, (1, HIDDEN), x, W1, dp0, dp1)


def _tc_h_g2(ap0, ap1, g1, dinv, b1, W2):
    def body(a0_ref, a1_ref, g1_ref, di_ref, b1_ref, w2_ref, g2_ref):
        dinv = di_ref[...]
        s = a0_ref[...] + a1_ref[...] - g1_ref[...]
        h = jnp.maximum(dinv * s + b1_ref[...], 0.0)
        z = jnp.dot(h, w2_ref[...], preferred_element_type=_f32)
        g2_ref[...] = dinv * z

    (g2,) = _tc_call(body, (1,), ap0, ap1, g1, dinv, b1, W2)
    return g2


def _tc_final(sp0, sp1, g2, dinv, b2):
    def body(s0_ref, s1_ref, g2_ref, di_ref, b2_ref, o_ref):
        s = s0_ref[...] + s1_ref[...] - g2_ref[...]
        o_ref[...] = di_ref[...] * s + b2_ref[...]

    (out,) = _tc_call(body, (1,), sp0, sp1, g2, dinv, b2)
    return out


def kernel(x, edge_index, W1, b1, W2, b2):
    ei = edge_index.astype(jnp.int32)
    src = jnp.pad(ei[0], (0, EPAD - N_EDGES), constant_values=PAD_NODE)
    dst = jnp.pad(ei[1], (0, EPAD - N_EDGES), constant_values=PAD_NODE)
    src3 = src.reshape(NW, NCH, CHUNK)
    dst3 = dst.reshape(NW, NCH, CHUNK)
    srcf = src.reshape(NW, NE_TILE)
    xp = jnp.pad(x, ((0, NPAD - N_NODES), (0, 0)))
    ones = jnp.ones((NPAD,), _f32)
    b1r = b1.reshape(1, HIDDEN)
    b2r = b2.reshape(1, 1)

    dp = _sc_scatter_scalar(ones, srcf, dst3, do_gather=False)
    dp0 = dp[0].reshape(NPAD, 1)
    dp1 = dp[1].reshape(NPAD, 1)
    dinv, g1 = _tc_deg_g1(xp, W1, dp0, dp1)
    ap = _sc_scatter_rows(g1, src3, dst3)
    g2 = _tc_h_g2(ap[0], ap[1], g1, dinv, b1r, W2)
    sp = _sc_scatter_scalar(g2.reshape(NPAD), srcf, dst3, do_gather=True)
    out = _tc_final(sp[0].reshape(NPAD, 1), sp[1].reshape(NPAD, 1),
                    g2, dinv, b2r)
    return out[:N_NODES, 0]


# speedup vs baseline: 57.1138x; 1.3729x over previous
import functools

import jax
import jax.numpy as jnp
import numpy as np
from jax import lax
from jax.experimental import pallas as pl
from jax.experimental.pallas import tpu as pltpu
from jax.experimental.pallas import tpu_sc as plsc

N_NODES = 10000
D_FEAT = 128
HIDDEN = 32
N_EDGES = 320000

NPAD = 10240
NC = 2
NS = 16
NW = NC * NS
CHUNK = 128
NCH = 80
NE_TILE = NCH * CHUNK
EPAD = NW * NE_TILE
ROWS_T = NPAD // NS
FIRE = 8

_f32 = jnp.float32


def _sc_mesh():
    return plsc.VectorSubcoreMesh(core_axis_name="c", subcore_axis_name="s")


_SC_PARAMS = pltpu.CompilerParams(use_tc_tiling_on_sc=False,
                                  needs_layout_passes=False)


def _sc_scatter_rows(v, src3, dst3):
    @functools.partial(
        pl.kernel,
        out_type=jax.ShapeDtypeStruct((NC, NPAD, HIDDEN), _f32),
        mesh=_sc_mesh(),
        compiler_params=_SC_PARAMS,
        scratch_types=[
            pltpu.VMEM((NCH, CHUNK), jnp.int32),
            pltpu.VMEM((NCH, CHUNK), jnp.int32),
            pltpu.VMEM((2, CHUNK, HIDDEN), _f32),
            pltpu.VMEM_SHARED((NPAD, HIDDEN), _f32),
            pltpu.SemaphoreType.DMA,
        ],
    )
    def k(v_hbm, src_hbm, dst_hbm, out_hbm, src_v, dst_v, bufs, acc_sh, gsem):
        cid = lax.axis_index("c")
        sid = lax.axis_index("s")
        wid = sid * NC + cid
        r0 = sid * ROWS_T
        pltpu.sync_copy(src_hbm.at[wid], src_v)
        pltpu.sync_copy(dst_hbm.at[wid], dst_v)
        pltpu.sync_copy(v_hbm.at[pl.ds(r0, ROWS_T)], acc_sh.at[pl.ds(r0, ROWS_T)])
        plsc.subcore_barrier()

        pltpu.make_async_copy(v_hbm.at[src_v.at[0]], bufs.at[0], gsem).start()

        def body(j, carry):
            slot = lax.rem(j, 2)
            nxt = lax.rem(j + 1, 2)

            @pl.when(j < NCH - 1)
            def _():
                pltpu.make_async_copy(
                    v_hbm.at[src_v.at[j + 1]], bufs.at[nxt], gsem).start()

            pltpu.make_async_copy(
                v_hbm.at[src_v.at[j]], bufs.at[slot], gsem).wait()
            pltpu.sync_copy(bufs.at[slot], acc_sh.at[dst_v.at[j]], add=True)
            return carry

        lax.fori_loop(0, NCH, body, 0, unroll=False)
        plsc.subcore_barrier()
        pltpu.sync_copy(acc_sh.at[pl.ds(r0, ROWS_T)],
                        out_hbm.at[cid, pl.ds(r0, ROWS_T)])

    return k(v, src3, dst3)


def _sc_scatter_scalar(v, srcf, dst3, do_gather):
    @functools.partial(
        pl.kernel,
        out_type=jax.ShapeDtypeStruct((NC, NPAD), _f32),
        mesh=_sc_mesh(),
        compiler_params=_SC_PARAMS,
        scratch_types=[
            pltpu.VMEM((NE_TILE,), jnp.int32),
            pltpu.VMEM((NCH, CHUNK), jnp.int32),
            pltpu.VMEM((NPAD,), _f32),
            pltpu.VMEM((NE_TILE,), _f32),
            pltpu.VMEM_SHARED((NPAD,), _f32),
            pltpu.SemaphoreType.DMA,
        ],
    )
    def k(v_hbm, src_hbm, dst_hbm, out_hbm, src_v, dst_v, g_v, val_v, acc_sh, ssem):
        cid = lax.axis_index("c")
        sid = lax.axis_index("s")
        wid = sid * NC + cid
        r0 = sid * ROWS_T
        pltpu.sync_copy(dst_hbm.at[wid], dst_v)
        pltpu.sync_copy(v_hbm.at[pl.ds(r0, ROWS_T)], acc_sh.at[pl.ds(r0, ROWS_T)])
        if do_gather:
            pltpu.sync_copy(src_hbm.at[wid], src_v)
            pltpu.sync_copy(v_hbm, g_v)

            def gbody(i, carry):
                idx = src_v[pl.ds(i * 16, 16)]
                val_v[pl.ds(i * 16, 16)] = plsc.load_gather(g_v, [idx])
                return carry

            lax.fori_loop(0, NE_TILE // 16, gbody, 0, unroll=4)
        else:
            ones = jnp.full((16,), 1.0, dtype=_f32)
            for i in range(CHUNK // 16):
                val_v[pl.ds(i * 16, 16)] = ones
        plsc.subcore_barrier()

        def group(g, carry):
            base = g * FIRE
            for t in range(FIRE):
                off = (base + t) * CHUNK if do_gather else 0
                pltpu.make_async_copy(
                    val_v.at[pl.ds(off, CHUNK)],
                    acc_sh.at[dst_v.at[base + t]], ssem).start(add=True)
            for t in range(FIRE):
                off = (base + t) * CHUNK if do_gather else 0
                pltpu.make_async_copy(
                    val_v.at[pl.ds(off, CHUNK)],
                    acc_sh.at[dst_v.at[base + t]], ssem).wait()
            return carry

        lax.fori_loop(0, NCH // FIRE, group, 0, unroll=False)
        plsc.subcore_barrier()
        pltpu.sync_copy(acc_sh.at[pl.ds(r0, ROWS_T)],
                        out_hbm.at[cid, pl.ds(r0, ROWS_T)])

    return k(v, srcf, dst3)


ROWS_B = 1280


def _tc_call(body, nout, *args):
    specs = []
    for a in args:
        if a.shape[0] == NPAD:
            specs.append(pl.BlockSpec((ROWS_B,) + a.shape[1:],
                                      lambda i: (i,) + (0,) * (a.ndim - 1)))
        else:
            specs.append(pl.BlockSpec(a.shape, lambda i: (0,) * a.ndim))
    out_shapes = [jax.ShapeDtypeStruct((NPAD, w), _f32) for w in nout]
    out_specs = [pl.BlockSpec((ROWS_B, w), lambda i: (i, 0)) for w in nout]
    res = pl.pallas_call(
        body,
        grid=(NPAD // ROWS_B,),
        in_specs=specs,
        out_specs=out_specs,
        out_shape=out_shapes,
    )(*args)
    return res


def _tc_deg_g1(x, W1, dp0, dp1):
    def body(x_ref, w_ref, d0_ref, d1_ref, dinv_ref, g1_ref):
        deg = d0_ref[...] + d1_ref[...] - 1.0
        dinv = lax.rsqrt(jnp.maximum(deg, 1.0))
        h0 = jnp.dot(x_ref[...], w_ref[...], preferred_element_type=_f32)
        dinv_ref[...] = dinv
        g1_ref[...] = dinv * h0

    return _tc_call(body, (1, HIDDEN), x, W1, dp0, dp1)


def _tc_h_g2(ap0, ap1, g1, dinv, b1, W2):
    def body(a0_ref, a1_ref, g1_ref, di_ref, b1_ref, w2_ref, g2_ref):
        dinv = di_ref[...]
        s = a0_ref[...] + a1_ref[...] - g1_ref[...]
        h = jnp.maximum(dinv * s + b1_ref[...], 0.0)
        z = jnp.dot(h, w2_ref[...], preferred_element_type=_f32)
        g2_ref[...] = dinv * z

    (g2,) = _tc_call(body, (1,), ap0, ap1, g1, dinv, b1, W2)
    return g2


def _tc_final(sp0, sp1, g2, dinv, b2):
    def body(s0_ref, s1_ref, g2_ref, di_ref, b2_ref, o_ref):
        s = s0_ref[...] + s1_ref[...] - g2_ref[...]
        o_ref[...] = di_ref[...] * s + b2_ref[...]

    (out,) = _tc_call(body, (1,), sp0, sp1, g2, dinv, b2)
    return out


_PAD_IDX = np.asarray(
    N_NODES + np.arange(EPAD - N_EDGES) % (NPAD - N_NODES), dtype=np.int32)


def kernel(x, edge_index, W1, b1, W2, b2):
    ei = edge_index.astype(jnp.int32)
    pad_idx = jnp.asarray(_PAD_IDX)
    src = jnp.concatenate([ei[0], pad_idx])
    dst = jnp.concatenate([ei[1], pad_idx])
    src3 = src.reshape(NW, NCH, CHUNK)
    dst3 = dst.reshape(NW, NCH, CHUNK)
    srcf = src.reshape(NW, NE_TILE)
    xp = jnp.pad(x, ((0, NPAD - N_NODES), (0, 0)))
    ones = jnp.ones((NPAD,), _f32)
    b1r = b1.reshape(1, HIDDEN)
    b2r = b2.reshape(1, 1)

    dp = _sc_scatter_scalar(ones, srcf, dst3, do_gather=False)
    dp0 = dp[0].reshape(NPAD, 1)
    dp1 = dp[1].reshape(NPAD, 1)
    dinv, g1 = _tc_deg_g1(xp, W1, dp0, dp1)
    ap = _sc_scatter_rows(g1, src3, dst3)
    g2 = _tc_h_g2(ap[0], ap[1], g1, dinv, b1r, W2)
    sp = _sc_scatter_scalar(g2.reshape(NPAD), srcf, dst3, do_gather=True)
    out = _tc_final(sp[0].reshape(NPAD, 1), sp[1].reshape(NPAD, 1),
                    g2, dinv, b2r)
    return out[:N_NODES, 0]
